# baseline (speedup 1.0000x reference)
import functools

import jax
import jax.numpy as jnp
from jax.experimental import pallas as pl
from jax.experimental.pallas import tpu as pltpu

_MARGIN = 1.0
_LEVEL = 131072.0
_KEY_CO = 512.0
_SBJ_CO_X = 1024.0
_SBJ_CO_Y = -128.0
_AUG = 128


def _prep_kernel(emb_ref, key_ref, sbj_ref,
                 xcat_ref, ycat_ref, sqx_ref, pval_ref):
    y = emb_ref[...]
    b, d = y.shape
    key = key_ref[0, :]
    sbj = sbj_ref[0, :]

    sqy = jnp.sum(y * y, axis=1)
    sqx_ref[...] = sqy[None, :]

    c = jax.lax.broadcasted_iota(jnp.int32, (b, _AUG), 1)
    key_b = key[:, None]
    sbj_b = sbj[:, None]

    xaug = (jnp.where(c == key_b, _KEY_CO, 0.0)
            + jnp.where(c == 32 + sbj_b, _SBJ_CO_X, 0.0)
            + jnp.where((c == 40) | (c == 41), 1.0, 0.0))
    xcat_ref[:, :d] = (-2.0 * y).astype(jnp.bfloat16)
    xcat_ref[:, d:] = xaug.astype(jnp.bfloat16)

    sqy_hi = sqy.astype(jnp.bfloat16).astype(jnp.float32)
    sqy_lo = sqy - sqy_hi
    yaug = (jnp.where(c == key_b, _KEY_CO, 0.0)
            + jnp.where(c == 32 + sbj_b, _SBJ_CO_Y, 0.0)
            + jnp.where(c == 40, sqy_hi[:, None], 0.0)
            + jnp.where(c == 41, sqy_lo[:, None], 0.0))
    ycat_ref[:, :d] = y.astype(jnp.bfloat16)
    ycat_ref[:, d:] = yaug.astype(jnp.bfloat16)

    kc = jax.lax.broadcasted_iota(jnp.int32, (32, b), 0)
    onehot = jnp.where(kc == key[None, :], 1.0, 0.0)
    hist = jnp.sum(onehot, axis=1, keepdims=True)
    cnt = jnp.sum(onehot * hist, axis=0)
    pval_ref[...] = jnp.where(cnt > 1.5, 1.0, 0.0)[None, :]


def _mine_kernel(nblocks, bi, xcat_ref, ycat_ref, sqx_ref, pval_ref,
                 out_ref, acc_ref):
    i = pl.program_id(0)
    xb = xcat_ref[...]

    g = jax.lax.dot_general(xb, ycat_ref[...], (((1,), (1,)), ((), ())),
                            preferred_element_type=jnp.float32)

    red_p = jnp.max(g, axis=1)
    red_n = jnp.min(g, axis=1)

    sqx_i = sqx_ref[0, pl.ds(i * bi, bi)]
    d_ap = jnp.sqrt(jnp.maximum(red_p - _LEVEL + sqx_i, 0.0))
    d_an = jnp.sqrt(jnp.maximum(red_n + _LEVEL + sqx_i, 0.0))

    valid = (pval_ref[0, pl.ds(i * bi, bi)] > 0.5) & (red_n < -65536.0)
    per_anchor = jnp.maximum(d_ap - d_an + _MARGIN, 0.0)
    psum = jnp.sum(jnp.where(valid, per_anchor, 0.0))
    pcnt = jnp.sum(valid.astype(jnp.float32))

    @pl.when(i == 0)
    def _init():
        acc_ref[0] = psum
        acc_ref[1] = pcnt

    @pl.when(i > 0)
    def _acc():
        acc_ref[0] += psum
        acc_ref[1] += pcnt

    @pl.when(i == nblocks - 1)
    def _finish():
        s = acc_ref[0]
        c = acc_ref[1]
        loss = jnp.where(c > 0.0, s / jnp.maximum(c, 1.0), 0.0)
        out_ref[...] = jnp.full((1, 1), loss, dtype=jnp.float32)


def kernel(emb, labels, sbj):
    b, d = emb.shape
    bi = 512
    nblocks = b // bi
    lbl32 = labels.astype(jnp.int32)
    sbj32 = sbj.astype(jnp.int32)
    key2 = (sbj32 * jnp.int32(4) + lbl32).reshape(1, b)
    sbj2 = sbj32.reshape(1, b)

    xcat, ycat, sqx, pval = pl.pallas_call(
        _prep_kernel,
        out_shape=[
            jax.ShapeDtypeStruct((b, d + _AUG), jnp.bfloat16),
            jax.ShapeDtypeStruct((b, d + _AUG), jnp.bfloat16),
            jax.ShapeDtypeStruct((1, b), jnp.float32),
            jax.ShapeDtypeStruct((1, b), jnp.float32),
        ],
    )(emb, key2, sbj2)

    out = pl.pallas_call(
        functools.partial(_mine_kernel, nblocks, bi),
        grid=(nblocks,),
        in_specs=[
            pl.BlockSpec((bi, d + _AUG), lambda i: (i, 0)),
            pl.BlockSpec((b, d + _AUG), lambda i: (0, 0)),
            pl.BlockSpec((1, b), lambda i: (0, 0)),
            pl.BlockSpec((1, b), lambda i: (0, 0)),
        ],
        out_specs=pl.BlockSpec((1, 1), lambda i: (0, 0)),
        out_shape=jax.ShapeDtypeStruct((1, 1), jnp.float32),
        scratch_shapes=[pltpu.SMEM((2,), jnp.float32)],
    )(xcat, ycat, sqx, pval)
    return out.reshape(())


# speedup vs baseline: 1.0317x; 1.0317x over previous
import functools

import jax
import jax.numpy as jnp
from jax.experimental import pallas as pl
from jax.experimental.pallas import tpu as pltpu

_MARGIN = 1.0
_LEVEL = 131072.0
_KEY_CO = 512.0
_SBJ_CO_X = 1024.0
_SBJ_CO_Y = -128.0
_AUG = 128


def _mine_kernel(nblocks, bi, xcat_ref, ycat_ref, sqx_ref, key_ref,
                 out_ref, acc_ref):
    i = pl.program_id(0)
    xb = xcat_ref[...]
    b = ycat_ref.shape[0]

    g = jax.lax.dot_general(xb, ycat_ref[...], (((1,), (1,)), ((), ())),
                            preferred_element_type=jnp.float32)

    red_p = jnp.max(g, axis=1)
    red_n = jnp.min(g, axis=1)

    sqx_i = sqx_ref[0, pl.ds(i * bi, bi)]
    d_ap = jnp.sqrt(jnp.maximum(red_p - _LEVEL + sqx_i, 0.0))
    d_an = jnp.sqrt(jnp.maximum(red_n + _LEVEL + sqx_i, 0.0))

    key = key_ref[0, :]
    key_i = key_ref[0, pl.ds(i * bi, bi)]
    kc = jax.lax.broadcasted_iota(jnp.int32, (32, b), 0)
    hist = jnp.sum(jnp.where(kc == key[None, :], 1.0, 0.0), axis=1)
    hc = jax.lax.broadcasted_iota(jnp.int32, (bi, 32), 1)
    cnt = jnp.sum(jnp.where(hc == key_i[:, None], hist[None, :], 0.0), axis=1)

    valid = (cnt > 1.5) & (red_n < -65536.0)
    per_anchor = jnp.maximum(d_ap - d_an + _MARGIN, 0.0)
    psum = jnp.sum(jnp.where(valid, per_anchor, 0.0))
    pcnt = jnp.sum(valid.astype(jnp.float32))

    @pl.when(i == 0)
    def _init():
        acc_ref[0] = psum
        acc_ref[1] = pcnt

    @pl.when(i > 0)
    def _acc():
        acc_ref[0] += psum
        acc_ref[1] += pcnt

    @pl.when(i == nblocks - 1)
    def _finish():
        s = acc_ref[0]
        c = acc_ref[1]
        loss = jnp.where(c > 0.0, s / jnp.maximum(c, 1.0), 0.0)
        out_ref[...] = jnp.full((1, 1), loss, dtype=jnp.float32)


def kernel(emb, labels, sbj):
    b, d = emb.shape
    bi = 512
    nblocks = b // bi
    lbl32 = labels.astype(jnp.int32)
    sbj32 = sbj.astype(jnp.int32)
    key = sbj32 * jnp.int32(4) + lbl32

    sqy = jnp.sum(emb * emb, axis=1)
    sqy_hi = sqy.astype(jnp.bfloat16).astype(jnp.float32)
    sqy_lo = sqy - sqy_hi
    c = jnp.arange(_AUG, dtype=jnp.int32)[None, :]
    key_b = key[:, None]
    sbj_b = sbj32[:, None]
    xaug = (jnp.where(c == key_b, _KEY_CO, 0.0)
            + jnp.where(c == 32 + sbj_b, _SBJ_CO_X, 0.0)
            + jnp.where((c == 40) | (c == 41), 1.0, 0.0))
    yaug = (jnp.where(c == key_b, _KEY_CO, 0.0)
            + jnp.where(c == 32 + sbj_b, _SBJ_CO_Y, 0.0)
            + jnp.where(c == 40, sqy_hi[:, None], 0.0)
            + jnp.where(c == 41, sqy_lo[:, None], 0.0))
    xcat = jnp.concatenate(
        [(-2.0 * emb).astype(jnp.bfloat16), xaug.astype(jnp.bfloat16)], axis=1)
    ycat = jnp.concatenate(
        [emb.astype(jnp.bfloat16), yaug.astype(jnp.bfloat16)], axis=1)

    out = pl.pallas_call(
        functools.partial(_mine_kernel, nblocks, bi),
        grid=(nblocks,),
        in_specs=[
            pl.BlockSpec((bi, d + _AUG), lambda i: (i, 0)),
            pl.BlockSpec((b, d + _AUG), lambda i: (0, 0)),
            pl.BlockSpec((1, b), lambda i: (0, 0)),
            pl.BlockSpec((1, b), lambda i: (0, 0)),
        ],
        out_specs=pl.BlockSpec((1, 1), lambda i: (0, 0)),
        out_shape=jax.ShapeDtypeStruct((1, 1), jnp.float32),
        scratch_shapes=[pltpu.SMEM((2,), jnp.float32)],
    )(xcat, ycat, sqy[None, :], key[None, :])
    return out.reshape(())


# speedup vs baseline: 1.1159x; 1.0816x over previous
import functools

import jax
import jax.numpy as jnp
from jax.experimental import pallas as pl
from jax.experimental.pallas import tpu as pltpu

_MARGIN = 1.0
_LEVEL = 131072.0
_KEY_CO = 512.0
_SBJ_CO_X = 1024.0
_SBJ_CO_Y = -128.0
_AUG = 128


def _triplet_kernel(nblocks, bi, emb_ref, key_ref, sbj_ref, out_ref,
                    xcat_ref, ycat_ref, aux_ref):
    y = emb_ref[...]
    b, d = y.shape
    key = key_ref[0, :]
    sbj = sbj_ref[0, :]

    sqy = jnp.sum(y * y, axis=1)
    aux_ref[0, :] = sqy

    c = jax.lax.broadcasted_iota(jnp.int32, (b, _AUG), 1)
    key_b = key[:, None]
    sbj_b = sbj[:, None]
    xaug = (jnp.where(c == key_b, _KEY_CO, 0.0)
            + jnp.where(c == 32 + sbj_b, _SBJ_CO_X, 0.0)
            + jnp.where((c == 40) | (c == 41), 1.0, 0.0))
    xcat_ref[:, :d] = (-2.0 * y).astype(jnp.bfloat16)
    xcat_ref[:, d:] = xaug.astype(jnp.bfloat16)

    sqy_hi = sqy.astype(jnp.bfloat16).astype(jnp.float32)
    sqy_lo = sqy - sqy_hi
    yaug = (jnp.where(c == key_b, _KEY_CO, 0.0)
            + jnp.where(c == 32 + sbj_b, _SBJ_CO_Y, 0.0)
            + jnp.where(c == 40, sqy_hi[:, None], 0.0)
            + jnp.where(c == 41, sqy_lo[:, None], 0.0))
    ycat_ref[:, :d] = y.astype(jnp.bfloat16)
    ycat_ref[:, d:] = yaug.astype(jnp.bfloat16)

    kc = jax.lax.broadcasted_iota(jnp.int32, (32, b), 0)
    onehot = jnp.where(kc == key[None, :], 1.0, 0.0)
    hist = jnp.sum(onehot, axis=1, keepdims=True)
    cnt = jnp.sum(onehot * hist, axis=0)
    aux_ref[1, :] = jnp.where(cnt > 1.5, 1.0, 0.0)

    def body(k, carry):
        ps, pc = carry
        xb = xcat_ref[pl.ds(k * bi, bi), :]
        g = jax.lax.dot_general(xb, ycat_ref[...], (((1,), (1,)), ((), ())),
                                preferred_element_type=jnp.float32)
        red_p = jnp.max(g, axis=1)
        red_n = jnp.min(g, axis=1)

        sqx_i = aux_ref[0, pl.ds(k * bi, bi)]
        d_ap = jnp.sqrt(jnp.maximum(red_p - _LEVEL + sqx_i, 0.0))
        d_an = jnp.sqrt(jnp.maximum(red_n + _LEVEL + sqx_i, 0.0))

        valid = (aux_ref[1, pl.ds(k * bi, bi)] > 0.5) & (red_n < -65536.0)
        per_anchor = jnp.maximum(d_ap - d_an + _MARGIN, 0.0)
        psum = jnp.sum(jnp.where(valid, per_anchor, 0.0))
        pcnt = jnp.sum(valid.astype(jnp.float32))
        return ps + psum, pc + pcnt

    s, cnt_v = jax.lax.fori_loop(0, nblocks, body, (0.0, 0.0))
    loss = jnp.where(cnt_v > 0.0, s / jnp.maximum(cnt_v, 1.0), 0.0)
    out_ref[...] = jnp.full((1, 1), loss, dtype=jnp.float32)


def kernel(emb, labels, sbj):
    b, d = emb.shape
    bi = 512
    nblocks = b // bi
    lbl32 = labels.astype(jnp.int32)
    sbj32 = sbj.astype(jnp.int32)
    key2 = (sbj32 * jnp.int32(4) + lbl32).reshape(1, b)
    sbj2 = sbj32.reshape(1, b)

    out = pl.pallas_call(
        functools.partial(_triplet_kernel, nblocks, bi),
        out_shape=jax.ShapeDtypeStruct((1, 1), jnp.float32),
        scratch_shapes=[
            pltpu.VMEM((b, d + _AUG), jnp.bfloat16),
            pltpu.VMEM((b, d + _AUG), jnp.bfloat16),
            pltpu.VMEM((2, b), jnp.float32),
        ],
    )(emb, key2, sbj2)
    return out.reshape(())


# speedup vs baseline: 1.2754x; 1.1429x over previous
import functools

import jax
import jax.numpy as jnp
from jax.experimental import pallas as pl
from jax.experimental.pallas import tpu as pltpu

_MARGIN = 1.0
_LEVEL = 131072.0
_KEY_CO = 512.0
_SBJ_CO_X = 1024.0
_SBJ_CO_Y = -128.0
_AUG = 128


def _triplet_kernel(nblocks, bi, emb_blk, emb_full, key_ref, sbj_ref,
                    out_ref, ycat_ref, hist_ref, acc_ref):
    i = pl.program_id(0)
    x = emb_blk[...]
    b, d = emb_full.shape

    key_i = key_ref[0, pl.ds(i * bi, bi)]
    sbj_i = sbj_ref[0, pl.ds(i * bi, bi)]

    @pl.when(i == 0)
    def _build_side_tables():
        y = emb_full[...]
        key = key_ref[0, :]
        sbj = sbj_ref[0, :]
        c = jax.lax.broadcasted_iota(jnp.int32, (b, _AUG), 1)
        kcol = jnp.where(c == key[:, None], _KEY_CO, 0.0)
        scol = jnp.where(c == 32 + sbj[:, None], _SBJ_CO_Y, 0.0)
        sqy = jnp.sum(y * y, axis=1)
        sqy_hi = sqy.astype(jnp.bfloat16).astype(jnp.float32)
        sqy_lo = sqy - sqy_hi
        qcol = (jnp.where(c == 40, sqy_hi[:, None], 0.0)
                + jnp.where(c == 41, sqy_lo[:, None], 0.0))
        ycat_ref[:, d:] = (kcol + scol + qcol).astype(jnp.bfloat16)
        ycat_ref[:, :d] = y.astype(jnp.bfloat16)
        kc = jax.lax.broadcasted_iota(jnp.int32, (32, b), 0)
        hist_ref[...] = jnp.sum(
            jnp.where(kc == key[None, :], 1.0, 0.0), axis=1, keepdims=True)

    cx = jax.lax.broadcasted_iota(jnp.int32, (bi, _AUG), 1)
    xaug = (jnp.where(cx == key_i[:, None], _KEY_CO, 0.0)
            + jnp.where(cx == 32 + sbj_i[:, None], _SBJ_CO_X, 0.0)
            + jnp.where((cx == 40) | (cx == 41), 1.0, 0.0))
    xcat = jnp.concatenate(
        [(-2.0 * x).astype(jnp.bfloat16), xaug.astype(jnp.bfloat16)], axis=1)

    dn = (((1,), (1,)), ((), ()))
    h = b // 2
    g1 = jax.lax.dot_general(xcat, ycat_ref[:h, :], dn,
                             preferred_element_type=jnp.float32)
    g2 = jax.lax.dot_general(xcat, ycat_ref[h:, :], dn,
                             preferred_element_type=jnp.float32)

    red_p = jnp.maximum(jnp.max(g1, axis=1), jnp.max(g2, axis=1))
    red_n = jnp.minimum(jnp.min(g1, axis=1), jnp.min(g2, axis=1))

    sqx = jnp.sum(x * x, axis=1)
    d_ap = jnp.sqrt(jnp.maximum(red_p - _LEVEL + sqx, 0.0))
    d_an = jnp.sqrt(jnp.maximum(red_n + _LEVEL + sqx, 0.0))

    hist = hist_ref[...]
    hc = jax.lax.broadcasted_iota(jnp.int32, (bi, 32), 1)
    cnt = jnp.sum(
        jnp.where(hc == key_i[:, None], hist[:, 0][None, :], 0.0), axis=1)

    valid = (cnt > 1.5) & (red_n < -65536.0)
    per_anchor = jnp.maximum(d_ap - d_an + _MARGIN, 0.0)
    psum = jnp.sum(jnp.where(valid, per_anchor, 0.0))
    pcnt = jnp.sum(valid.astype(jnp.float32))

    @pl.when(i == 0)
    def _init():
        acc_ref[0] = psum
        acc_ref[1] = pcnt

    @pl.when(i > 0)
    def _acc():
        acc_ref[0] += psum
        acc_ref[1] += pcnt

    @pl.when(i == nblocks - 1)
    def _finish():
        s = acc_ref[0]
        c = acc_ref[1]
        loss = jnp.where(c > 0.0, s / jnp.maximum(c, 1.0), 0.0)
        out_ref[...] = jnp.full((1, 1), loss, dtype=jnp.float32)


def kernel(emb, labels, sbj):
    b, d = emb.shape
    bi = 512
    nblocks = b // bi
    lbl32 = labels.astype(jnp.int32)
    sbj32 = sbj.astype(jnp.int32)
    key2 = (sbj32 * jnp.int32(4) + lbl32).reshape(1, b)
    sbj2 = sbj32.reshape(1, b)

    out = pl.pallas_call(
        functools.partial(_triplet_kernel, nblocks, bi),
        grid=(nblocks,),
        in_specs=[
            pl.BlockSpec((bi, d), lambda i: (i, 0)),
            pl.BlockSpec((b, d), lambda i: (0, 0)),
            pl.BlockSpec((1, b), lambda i: (0, 0)),
            pl.BlockSpec((1, b), lambda i: (0, 0)),
        ],
        out_specs=pl.BlockSpec((1, 1), lambda i: (0, 0)),
        out_shape=jax.ShapeDtypeStruct((1, 1), jnp.float32),
        scratch_shapes=[
            pltpu.VMEM((b, d + _AUG), jnp.bfloat16),
            pltpu.VMEM((32, 1), jnp.float32),
            pltpu.SMEM((2,), jnp.float32),
        ],
    )(emb, emb, key2, sbj2)
    return out.reshape(())


# speedup vs baseline: 1.3381x; 1.0492x over previous
import functools

import jax
import jax.numpy as jnp
from jax.experimental import pallas as pl
from jax.experimental.pallas import tpu as pltpu

_MARGIN = 1.0
_LEVEL = 131072.0
_KEY_CO = 512.0
_SBJ_CO_X = 1024.0
_SBJ_CO_Y = -128.0
_AUG = 128


def _triplet_kernel(nblocks, bi, emb_blk, emb_full, key_ref, sbj_ref,
                    out_ref, ycat_ref, hist_ref, acc_ref):
    i = pl.program_id(0)
    x = emb_blk[...]
    b, d = emb_full.shape

    key_i = key_ref[0, pl.ds(i * bi, bi)]
    sbj_i = sbj_ref[0, pl.ds(i * bi, bi)]

    @pl.when(i == 0)
    def _build_side_tables():
        y = emb_full[...]
        key = key_ref[0, :]
        sbj = sbj_ref[0, :]
        c = jax.lax.broadcasted_iota(jnp.int32, (b, _AUG), 1)
        kcol = jnp.where(c == key[:, None], _KEY_CO, 0.0)
        scol = jnp.where(c == 32 + sbj[:, None], _SBJ_CO_Y, 0.0)
        sqy = jnp.sum(y * y, axis=1)
        sqy_hi = sqy.astype(jnp.bfloat16).astype(jnp.float32)
        sqy_lo = sqy - sqy_hi
        qcol = (jnp.where(c == 40, sqy_hi[:, None], 0.0)
                + jnp.where(c == 41, sqy_lo[:, None], 0.0))
        ycat_ref[:, d:] = (kcol + scol + qcol).astype(jnp.bfloat16)
        ycat_ref[:, :d] = y.astype(jnp.bfloat16)
        kc = jax.lax.broadcasted_iota(jnp.int32, (32, b), 0)
        hist_ref[...] = jnp.sum(
            jnp.where(kc == key[None, :], 1.0, 0.0), axis=1, keepdims=True)

    cx = jax.lax.broadcasted_iota(jnp.int32, (bi, _AUG), 1)
    xaug = (jnp.where(cx == key_i[:, None], _KEY_CO, 0.0)
            + jnp.where(cx == 32 + sbj_i[:, None], _SBJ_CO_X, 0.0)
            + jnp.where((cx == 40) | (cx == 41), 1.0, 0.0))
    xcat = jnp.concatenate(
        [(-2.0 * x).astype(jnp.bfloat16), xaug.astype(jnp.bfloat16)], axis=1)

    dn = (((1,), (1,)), ((), ()))
    h = b // 2
    g1 = jax.lax.dot_general(xcat, ycat_ref[:h, :], dn,
                             preferred_element_type=jnp.float32)
    g2 = jax.lax.dot_general(xcat, ycat_ref[h:, :], dn,
                             preferred_element_type=jnp.float32)

    red_p = jnp.maximum(jnp.max(g1, axis=1), jnp.max(g2, axis=1))
    red_n = jnp.minimum(jnp.min(g1, axis=1), jnp.min(g2, axis=1))

    sqx = jnp.sum(x * x, axis=1)
    d_ap = jnp.sqrt(jnp.maximum(red_p - _LEVEL + sqx, 0.0))
    d_an = jnp.sqrt(jnp.maximum(red_n + _LEVEL + sqx, 0.0))

    hist = hist_ref[...]
    hc = jax.lax.broadcasted_iota(jnp.int32, (bi, 32), 1)
    cnt = jnp.sum(
        jnp.where(hc == key_i[:, None], hist[:, 0][None, :], 0.0), axis=1)

    valid = (cnt > 1.5) & (red_n < -65536.0)
    per_anchor = jnp.maximum(d_ap - d_an + _MARGIN, 0.0)
    psum = jnp.sum(jnp.where(valid, per_anchor, 0.0))
    pcnt = jnp.sum(valid.astype(jnp.float32))

    @pl.when(i == 0)
    def _init():
        acc_ref[0] = psum
        acc_ref[1] = pcnt

    @pl.when(i > 0)
    def _acc():
        acc_ref[0] += psum
        acc_ref[1] += pcnt

    @pl.when(i == nblocks - 1)
    def _finish():
        s = acc_ref[0]
        c = acc_ref[1]
        loss = jnp.where(c > 0.0, s / jnp.maximum(c, 1.0), 0.0)
        out_ref[...] = jnp.full((1, 1), loss, dtype=jnp.float32)


def kernel(emb, labels, sbj):
    b, d = emb.shape
    bi = 1024
    nblocks = b // bi
    lbl32 = labels.astype(jnp.int32)
    sbj32 = sbj.astype(jnp.int32)
    key2 = (sbj32 * jnp.int32(4) + lbl32).reshape(1, b)
    sbj2 = sbj32.reshape(1, b)

    out = pl.pallas_call(
        functools.partial(_triplet_kernel, nblocks, bi),
        grid=(nblocks,),
        in_specs=[
            pl.BlockSpec((bi, d), lambda i: (i, 0)),
            pl.BlockSpec((b, d), lambda i: (0, 0)),
            pl.BlockSpec((1, b), lambda i: (0, 0)),
            pl.BlockSpec((1, b), lambda i: (0, 0)),
        ],
        out_specs=pl.BlockSpec((1, 1), lambda i: (0, 0)),
        out_shape=jax.ShapeDtypeStruct((1, 1), jnp.float32),
        scratch_shapes=[
            pltpu.VMEM((b, d + _AUG), jnp.bfloat16),
            pltpu.VMEM((32, 1), jnp.float32),
            pltpu.SMEM((2,), jnp.float32),
        ],
    )(emb, emb, key2, sbj2)
    return out.reshape(())


# speedup vs baseline: 1.3964x; 1.0436x over previous
import functools

import jax
import jax.numpy as jnp
from jax.experimental import pallas as pl
from jax.experimental.pallas import tpu as pltpu

_MARGIN = 1.0
_LEVEL = 131072.0
_KEY_CO = 512.0
_SBJ_CO_X = 1024.0
_SBJ_CO_Y = -128.0
_AUG = 128


def _triplet_kernel(nblocks, bi, emb_blk, emb_full, key_ref, sbj_ref,
                    out_ref, ycat_ref, hist_ref, acc_ref):
    i = pl.program_id(0)
    x = emb_blk[...]
    b, d = emb_full.shape

    key_i = key_ref[0, pl.ds(i * bi, bi)]
    sbj_i = sbj_ref[0, pl.ds(i * bi, bi)]

    @pl.when(i == 0)
    def _build_side_tables():
        y = emb_full[...]
        key = key_ref[0, :]
        sbj = sbj_ref[0, :]
        c = jax.lax.broadcasted_iota(jnp.int32, (b, _AUG), 1)
        kcol = jnp.where(c == key[:, None], _KEY_CO, 0.0)
        scol = jnp.where(c == 32 + sbj[:, None], _SBJ_CO_Y, 0.0)
        sqy = jnp.sum(y * y, axis=1)
        sqy_hi = sqy.astype(jnp.bfloat16).astype(jnp.float32)
        sqy_lo = sqy - sqy_hi
        qcol = (jnp.where(c == 40, sqy_hi[:, None], 0.0)
                + jnp.where(c == 41, sqy_lo[:, None], 0.0))
        ycat_ref[:, d:] = (kcol + scol + qcol).astype(jnp.bfloat16)
        ycat_ref[:, :d] = y.astype(jnp.bfloat16)
        kc = jax.lax.broadcasted_iota(jnp.int32, (32, b), 0)
        hist_ref[...] = jnp.sum(
            jnp.where(kc == key[None, :], 1.0, 0.0), axis=1, keepdims=True)

    cx = jax.lax.broadcasted_iota(jnp.int32, (bi, _AUG), 1)
    xaug = (jnp.where(cx == key_i[:, None], _KEY_CO, 0.0)
            + jnp.where(cx == 32 + sbj_i[:, None], _SBJ_CO_X, 0.0)
            + jnp.where((cx == 40) | (cx == 41), 1.0, 0.0))
    xcat = jnp.concatenate(
        [(-2.0 * x).astype(jnp.bfloat16), xaug.astype(jnp.bfloat16)], axis=1)

    dn = (((1,), (1,)), ((), ()))
    h = b // 2
    g1 = jax.lax.dot_general(xcat, ycat_ref[:h, :], dn,
                             preferred_element_type=jnp.float32)
    g2 = jax.lax.dot_general(xcat, ycat_ref[h:, :], dn,
                             preferred_element_type=jnp.float32)

    red_p = jnp.maximum(jnp.max(g1, axis=1), jnp.max(g2, axis=1))
    red_n = jnp.minimum(jnp.min(g1, axis=1), jnp.min(g2, axis=1))

    sqx = jnp.sum(x * x, axis=1)
    d_ap = jnp.sqrt(jnp.maximum(red_p - _LEVEL + sqx, 0.0))
    d_an = jnp.sqrt(jnp.maximum(red_n + _LEVEL + sqx, 0.0))

    hist = hist_ref[...]
    hc = jax.lax.broadcasted_iota(jnp.int32, (bi, 32), 1)
    cnt = jnp.sum(
        jnp.where(hc == key_i[:, None], hist[:, 0][None, :], 0.0), axis=1)

    valid = (cnt > 1.5) & (red_n < -65536.0)
    per_anchor = jnp.maximum(d_ap - d_an + _MARGIN, 0.0)
    psum = jnp.sum(jnp.where(valid, per_anchor, 0.0))
    pcnt = jnp.sum(valid.astype(jnp.float32))

    @pl.when(i == 0)
    def _init():
        acc_ref[0] = psum
        acc_ref[1] = pcnt

    @pl.when(i > 0)
    def _acc():
        acc_ref[0] += psum
        acc_ref[1] += pcnt

    @pl.when(i == nblocks - 1)
    def _finish():
        s = acc_ref[0]
        c = acc_ref[1]
        loss = jnp.where(c > 0.0, s / jnp.maximum(c, 1.0), 0.0)
        out_ref[...] = jnp.full((1, 1), loss, dtype=jnp.float32)


def kernel(emb, labels, sbj):
    b, d = emb.shape
    bi = 2048
    nblocks = b // bi
    lbl32 = labels.astype(jnp.int32)
    sbj32 = sbj.astype(jnp.int32)
    key2 = (sbj32 * jnp.int32(4) + lbl32).reshape(1, b)
    sbj2 = sbj32.reshape(1, b)

    out = pl.pallas_call(
        functools.partial(_triplet_kernel, nblocks, bi),
        grid=(nblocks,),
        in_specs=[
            pl.BlockSpec((bi, d), lambda i: (i, 0)),
            pl.BlockSpec((b, d), lambda i: (0, 0)),
            pl.BlockSpec((1, b), lambda i: (0, 0)),
            pl.BlockSpec((1, b), lambda i: (0, 0)),
        ],
        out_specs=pl.BlockSpec((1, 1), lambda i: (0, 0)),
        out_shape=jax.ShapeDtypeStruct((1, 1), jnp.float32),
        scratch_shapes=[
            pltpu.VMEM((b, d + _AUG), jnp.bfloat16),
            pltpu.VMEM((32, 1), jnp.float32),
            pltpu.SMEM((2,), jnp.float32),
        ],
    )(emb, emb, key2, sbj2)
    return out.reshape(())


# speedup vs baseline: 1.5818x; 1.1327x over previous
import functools

import jax
import jax.numpy as jnp
from jax.experimental import pallas as pl
from jax.experimental.pallas import tpu as pltpu

_MARGIN = 1.0
_LEVEL = 131072.0
_KEY_CO = 512.0
_SBJ_CO_X = 1024.0
_SBJ_CO_Y = -128.0
_AUG = 128
_NCHUNK = 8


def _triplet_kernel(emb_ref, key_ref, sbj_ref, out_ref, ycat_ref):
    y = emb_ref[...]
    b, d = y.shape
    key = key_ref[0, :]
    sbj = sbj_ref[0, :]

    sqy = jnp.sum(y * y, axis=1)
    c = jax.lax.broadcasted_iota(jnp.int32, (b, _AUG), 1)
    key_b = key[:, None]
    sbj_b = sbj[:, None]

    sqy_hi = sqy.astype(jnp.bfloat16).astype(jnp.float32)
    sqy_lo = sqy - sqy_hi
    yaug = (jnp.where(c == key_b, _KEY_CO, 0.0)
            + jnp.where(c == 32 + sbj_b, _SBJ_CO_Y, 0.0)
            + jnp.where(c == 40, sqy_hi[:, None], 0.0)
            + jnp.where(c == 41, sqy_lo[:, None], 0.0))
    ycat_ref[:, :d] = y.astype(jnp.bfloat16)
    ycat_ref[:, d:] = yaug.astype(jnp.bfloat16)

    xaug = (jnp.where(c == key_b, _KEY_CO, 0.0)
            + jnp.where(c == 32 + sbj_b, _SBJ_CO_X, 0.0)
            + jnp.where((c == 40) | (c == 41), 1.0, 0.0))
    xcat = jnp.concatenate(
        [(-2.0 * y).astype(jnp.bfloat16), xaug.astype(jnp.bfloat16)], axis=1)

    dn = (((1,), (1,)), ((), ()))
    ch = b // _NCHUNK
    maxs = []
    mins = []
    for k in range(_NCHUNK):
        gk = jax.lax.dot_general(xcat, ycat_ref[k * ch:(k + 1) * ch, :], dn,
                                 preferred_element_type=jnp.float32)
        maxs.append(jnp.max(gk, axis=1))
        mins.append(jnp.min(gk, axis=1))
    red_p = functools.reduce(jnp.maximum, maxs)
    red_n = functools.reduce(jnp.minimum, mins)

    d_ap = jnp.sqrt(jnp.maximum(red_p - _LEVEL + sqy, 0.0))
    d_an = jnp.sqrt(jnp.maximum(red_n + _LEVEL + sqy, 0.0))

    kc = jax.lax.broadcasted_iota(jnp.int32, (32, b), 0)
    onehot = jnp.where(kc == key[None, :], 1.0, 0.0)
    hist = jnp.sum(onehot, axis=1, keepdims=True)
    cnt = jnp.sum(onehot * hist, axis=0)

    valid = (cnt > 1.5) & (red_n < -65536.0)
    per_anchor = jnp.maximum(d_ap - d_an + _MARGIN, 0.0)
    s = jnp.sum(jnp.where(valid, per_anchor, 0.0))
    cnt_v = jnp.sum(valid.astype(jnp.float32))
    loss = jnp.where(cnt_v > 0.0, s / jnp.maximum(cnt_v, 1.0), 0.0)
    out_ref[...] = jnp.full((1, 1), loss, dtype=jnp.float32)


def kernel(emb, labels, sbj):
    b, d = emb.shape
    lbl32 = labels.astype(jnp.int32)
    sbj32 = sbj.astype(jnp.int32)
    key2 = (sbj32 * jnp.int32(4) + lbl32).reshape(1, b)
    sbj2 = sbj32.reshape(1, b)

    out = pl.pallas_call(
        _triplet_kernel,
        out_shape=jax.ShapeDtypeStruct((1, 1), jnp.float32),
        scratch_shapes=[
            pltpu.VMEM((b, d + _AUG), jnp.bfloat16),
        ],
    )(emb, key2, sbj2)
    return out.reshape(())
